# baseline (speedup 1.0000x reference)
import functools

import jax
import jax.numpy as jnp
from jax import lax
from jax.experimental import pallas as pl
from jax.experimental.pallas import tpu as pltpu
from jax.experimental.pallas import tpu_sc as plsc

B, S, E = 1024, 200, 64
NW = 32
ROWS_PER_W = (B * S) // NW
CHUNK = 200
NCHUNK = ROWS_PER_W // CHUNK
VECS = (CHUNK * E) // 16


def _sc_embed(x_flat, vocab_table, pos_table):
    mesh = plsc.VectorSubcoreMesh(core_axis_name="c", subcore_axis_name="s")

    @functools.partial(
        pl.kernel,
        out_type=jax.ShapeDtypeStruct((B * S, E), jnp.float32),
        scratch_types=[
            pltpu.VMEM((ROWS_PER_W,), jnp.int32),
            pltpu.VMEM((CHUNK, E), jnp.float32),
            pltpu.VMEM((CHUNK, E), jnp.float32),
            pltpu.SemaphoreType.DMA,
        ],
        mesh=mesh,
        compiler_params=pltpu.CompilerParams(use_tc_tiling_on_sc=False),
    )
    def k(x_hbm, vocab_hbm, pos_hbm, out_hbm, idx_v, pos_v, buf, sem):
        wid = lax.axis_index("s") * 2 + lax.axis_index("c")
        pltpu.sync_copy(x_hbm.at[wid], idx_v)
        pltpu.sync_copy(pos_hbm.at[pl.ds(0, CHUNK)], pos_v)

        def chunk_body(ch, _):
            base = ch * CHUNK
            pltpu.async_copy(
                vocab_hbm.at[idx_v.at[pl.ds(base, CHUNK)]], buf, sem
            ).wait()

            def add_body(i, _):
                for v in range(E // 16):
                    sl = pl.ds(v * 16, 16)
                    buf[i, sl] = buf[i, sl] + pos_v[i, sl]
                return ()

            lax.fori_loop(0, CHUNK, add_body, ())

            pltpu.sync_copy(buf, out_hbm.at[pl.ds(wid * ROWS_PER_W + base, CHUNK)])
            return ()

        lax.fori_loop(0, NCHUNK, chunk_body, ())

    return k(x_flat, vocab_table, pos_table)


def kernel(x, vocab_table, pos_table):
    x_flat = x.reshape(NW, ROWS_PER_W)
    out = _sc_embed(x_flat, vocab_table, pos_table)
    return out.reshape(B, S, E)


# speedup vs baseline: 1.0457x; 1.0457x over previous
import functools

import jax
import jax.numpy as jnp
from jax import lax
from jax.experimental import pallas as pl
from jax.experimental.pallas import tpu as pltpu
from jax.experimental.pallas import tpu_sc as plsc

B, S, E = 1024, 200, 64
NW = 32
ROWS_PER_W = B // NW
NB = 8
NGROUP = ROWS_PER_W // NB


def kernel(x, vocab_table, pos_table):
    mesh = plsc.VectorSubcoreMesh(core_axis_name="c", subcore_axis_name="s")

    @functools.partial(
        pl.kernel,
        out_type=jax.ShapeDtypeStruct((B, S, E), jnp.float32),
        scratch_types=[
            pltpu.VMEM((ROWS_PER_W, S), jnp.int32),
            pltpu.VMEM((S, E), jnp.float32),
            pltpu.VMEM((NB, S, E), jnp.float32),
            pltpu.SemaphoreType.DMA((NB,)),
            pltpu.SemaphoreType.DMA((NB,)),
        ],
        mesh=mesh,
        compiler_params=pltpu.CompilerParams(use_tc_tiling_on_sc=False),
    )
    def k(x_hbm, vocab_hbm, pos_hbm, out_hbm, idx_v, pos_v, buf, gsems, ssems):
        wid = lax.axis_index("s") * 2 + lax.axis_index("c")
        base_row = wid * ROWS_PER_W
        pltpu.sync_copy(x_hbm.at[pl.ds(base_row, ROWS_PER_W)], idx_v)
        pltpu.sync_copy(pos_hbm.at[pl.ds(0, S)], pos_v)

        def gather(ch, b):
            return pltpu.make_async_copy(
                vocab_hbm.at[idx_v.at[ch]], buf.at[b], gsems.at[b]
            )

        def store(b, ch):
            return pltpu.make_async_copy(
                buf.at[b], out_hbm.at[base_row + ch], ssems.at[b]
            )

        for b in range(NB):
            gather(b, b).start()

        def group(g, _):
            for b in range(NB):
                ch = g * NB + b
                gather(ch, b).wait()

                def add_body(i, _, b=b):
                    for v in range(E // 16):
                        sl = pl.ds(v * 16, 16)
                        buf[b, i, sl] = buf[b, i, sl] + pos_v[i, sl]
                    return ()

                lax.fori_loop(0, S, add_body, ())
                store(b, ch).start()

            @pl.when(g < NGROUP - 1)
            def _():
                for b in range(NB):
                    store(b, g * NB + b).wait()
                    gather((g + 1) * NB + b, b).start()

            return ()

        lax.fori_loop(0, NGROUP, group, ())
        for b in range(NB):
            store(b, (NGROUP - 1) * NB + b).wait()

    return k(x, vocab_table, pos_table)
